# Initial kernel scaffold; baseline (speedup 1.0000x reference)
import jax
import jax.numpy as jnp
from jax import lax
from jax.experimental import pallas as pl
from jax.experimental.pallas import tpu as pltpu
from jax.experimental.pallas import tpu_sc as plsc

N_NODES = 10000
N_EDGES = 320000
D_NODE = 128
D_EDGE = 16
D_OUT = 128

LANES = 16
CW = 16
NC = 2
NS = 16
NW = NC * NS
EPW = N_EDGES // NW
CHUNK = 80
NCHUNKS = EPW // CHUNK
ROWS_PER_TILE = N_NODES // NS
ZROWS = 125

_DOT_KW = dict(preferred_element_type=jnp.float32,
               precision=lax.Precision.HIGHEST)


def _pq_body(x_ref, ws_ref, wd_ref, p_ref, q_ref):
    x = x_ref[...]
    p_ref[...] = lax.dot_general(x, ws_ref[...], (((1,), (0,)), ((), ())),
                                 **_DOT_KW)
    q_ref[...] = lax.dot_general(x, wd_ref[...], (((1,), (0,)), ((), ())),
                                 **_DOT_KW)


_BN = 1000
_pq_call = pl.pallas_call(
    _pq_body,
    grid=(N_NODES // _BN,),
    in_specs=[
        pl.BlockSpec((_BN, D_NODE), lambda i: (i, 0)),
        pl.BlockSpec((D_NODE, D_OUT), lambda i: (0, 0)),
        pl.BlockSpec((D_NODE, D_OUT), lambda i: (0, 0)),
    ],
    out_specs=[
        pl.BlockSpec((_BN, D_OUT), lambda i: (i, 0)),
        pl.BlockSpec((_BN, D_OUT), lambda i: (i, 0)),
    ],
    out_shape=[
        jax.ShapeDtypeStruct((N_NODES, D_OUT), jnp.float32),
        jax.ShapeDtypeStruct((N_NODES, D_OUT), jnp.float32),
    ],
)


def _e1_body(ef_ref, we_ref, b1_ref, e1_ref):
    e1_ref[...] = lax.dot_general(ef_ref[...], we_ref[...],
                                  (((1,), (0,)), ((), ())),
                                  **_DOT_KW) + b1_ref[...]


_BE = 4000
_e1_call = pl.pallas_call(
    _e1_body,
    grid=(N_EDGES // _BE,),
    in_specs=[
        pl.BlockSpec((_BE, D_EDGE), lambda i: (i, 0)),
        pl.BlockSpec((D_EDGE, D_OUT), lambda i: (0, 0)),
        pl.BlockSpec((1, D_OUT), lambda i: (0, 0)),
    ],
    out_specs=pl.BlockSpec((_BE, D_OUT), lambda i: (i, 0)),
    out_shape=jax.ShapeDtypeStruct((N_EDGES, D_OUT), jnp.float32),
)


def _sc_edge_body(p_hbm, q_hbm, e1_hbm, src_hbm, dst_hbm,
                  agg_out, cnt_out,
                  idx_s, idx_d, buf_p, buf_q, buf_e, ones_v, zrow_v, zcnt_v,
                  agg_sh, cnt_sh, sem_p, sem_q, sem_e):
    cid = lax.axis_index("c")
    sid = lax.axis_index("s")
    wid = sid * NC + cid

    zf = jnp.zeros((LANES,), jnp.float32)
    onef = jnp.ones((LANES,), jnp.float32)

    def _zrow(r, carry):
        for c in range(D_OUT // LANES):
            zrow_v[r, pl.ds(c * LANES, LANES)] = zf
        return carry

    lax.fori_loop(0, ZROWS, _zrow, 0)

    def _zcnt(r, carry):
        zcnt_v[r, pl.ds(0, LANES)] = zf
        return carry

    lax.fori_loop(0, ROWS_PER_TILE, _zcnt, 0)

    def _ones(r, carry):
        ones_v[r, pl.ds(0, LANES)] = onef
        return carry

    lax.fori_loop(0, CHUNK, _ones, 0)

    base_row = sid * ROWS_PER_TILE
    for k in range(ROWS_PER_TILE // ZROWS):
        pltpu.sync_copy(zrow_v, agg_sh.at[pl.ds(base_row + k * ZROWS, ZROWS)])
    pltpu.sync_copy(zcnt_v, cnt_sh.at[pl.ds(base_row, ROWS_PER_TILE)])
    plsc.subcore_barrier()

    ebase = wid * EPW

    def _chunk(ci, carry):
        off = pl.multiple_of(ebase + ci * CHUNK, CHUNK)
        cp_e = pltpu.async_copy(e1_hbm.at[pl.ds(off, CHUNK)], buf_e, sem_e)
        pltpu.sync_copy(src_hbm.at[pl.ds(off, CHUNK)], idx_s)
        pltpu.sync_copy(dst_hbm.at[pl.ds(off, CHUNK)], idx_d)
        cp_p = pltpu.async_copy(p_hbm.at[idx_s], buf_p, sem_p)
        cp_q = pltpu.async_copy(q_hbm.at[idx_d], buf_q, sem_q)
        cp_e.wait()
        cp_p.wait()
        cp_q.wait()

        def _row(r, rcarry):
            for c in range(D_OUT // LANES):
                sl = pl.ds(c * LANES, LANES)
                v = buf_p[r, sl] + buf_q[r, sl] + buf_e[r, sl]
                buf_e[r, sl] = jnp.maximum(v, 0.0)
            return rcarry

        lax.fori_loop(0, CHUNK, _row, 0)

        pltpu.sync_copy(buf_e, agg_sh.at[idx_d], add=True)
        pltpu.sync_copy(ones_v, cnt_sh.at[idx_d], add=True)
        return carry

    lax.fori_loop(0, NCHUNKS, _chunk, 0)

    plsc.subcore_barrier()

    pltpu.sync_copy(agg_sh.at[pl.ds(base_row, ROWS_PER_TILE)],
                    agg_out.at[cid, pl.ds(base_row, ROWS_PER_TILE)])
    pltpu.sync_copy(cnt_sh.at[pl.ds(base_row, ROWS_PER_TILE)],
                    cnt_out.at[cid, pl.ds(base_row, ROWS_PER_TILE)])


_sc_edge = pl.kernel(
    _sc_edge_body,
    mesh=plsc.VectorSubcoreMesh(core_axis_name="c", subcore_axis_name="s"),
    out_type=[
        jax.ShapeDtypeStruct((NC, N_NODES, D_OUT), jnp.float32),
        jax.ShapeDtypeStruct((NC, N_NODES, CW), jnp.float32),
    ],
    scratch_types=[
        pltpu.VMEM((CHUNK,), jnp.int32),
        pltpu.VMEM((CHUNK,), jnp.int32),
        pltpu.VMEM((CHUNK, D_OUT), jnp.float32),
        pltpu.VMEM((CHUNK, D_OUT), jnp.float32),
        pltpu.VMEM((CHUNK, D_OUT), jnp.float32),
        pltpu.VMEM((CHUNK, CW), jnp.float32),
        pltpu.VMEM((ZROWS, D_OUT), jnp.float32),
        pltpu.VMEM((ROWS_PER_TILE, CW), jnp.float32),
        pltpu.VMEM_SHARED((N_NODES, D_OUT), jnp.float32),
        pltpu.VMEM_SHARED((N_NODES, CW), jnp.float32),
        pltpu.SemaphoreType.DMA,
        pltpu.SemaphoreType.DMA,
        pltpu.SemaphoreType.DMA,
    ],
)


def _out_body(a0_ref, a1_ref, c0_ref, c1_ref, w2_ref, b2_ref, o_ref):
    agg = a0_ref[...] + a1_ref[...]
    deg = c0_ref[...][:, :1] + c1_ref[...][:, :1]
    o_ref[...] = lax.dot_general(agg, w2_ref[...], (((1,), (0,)), ((), ())),
                                 **_DOT_KW) + deg * b2_ref[...]


_out_call = pl.pallas_call(
    _out_body,
    grid=(N_NODES // _BN,),
    in_specs=[
        pl.BlockSpec((_BN, D_OUT), lambda i: (i, 0)),
        pl.BlockSpec((_BN, D_OUT), lambda i: (i, 0)),
        pl.BlockSpec((_BN, CW), lambda i: (i, 0)),
        pl.BlockSpec((_BN, CW), lambda i: (i, 0)),
        pl.BlockSpec((D_OUT, D_OUT), lambda i: (0, 0)),
        pl.BlockSpec((1, D_OUT), lambda i: (0, 0)),
    ],
    out_specs=pl.BlockSpec((_BN, D_OUT), lambda i: (i, 0)),
    out_shape=jax.ShapeDtypeStruct((N_NODES, D_OUT), jnp.float32),
)


def kernel(node_feats, edge_index, edge_feats, W1, b1, W2, b2):
    src = edge_index[0].astype(jnp.int32)
    dst = edge_index[1].astype(jnp.int32)
    p, q = _pq_call(node_feats, W1[:D_NODE], W1[D_NODE:2 * D_NODE])
    e1 = _e1_call(edge_feats, W1[2 * D_NODE:], b1.reshape(1, D_OUT))
    agg2, cnt2 = _sc_edge(p, q, e1, src, dst)
    out = _out_call(agg2[0], agg2[1], cnt2[0], cnt2[1],
                    W2, b2.reshape(1, D_OUT))
    return out


# speedup vs baseline: 3.5476x; 3.5476x over previous
import jax
import jax.numpy as jnp
from jax import lax
from jax.experimental import pallas as pl
from jax.experimental.pallas import tpu as pltpu
from jax.experimental.pallas import tpu_sc as plsc

N_NODES = 10000
N_EDGES = 320000
D_NODE = 128
D_EDGE = 16
D_OUT = 128

LANES = 16
CW = 16
NC = 2
NS = 16
NW = NC * NS
EPW = N_EDGES // NW
CHUNK = 80
NCHUNKS = EPW // CHUNK
N_PAD = 10240
ROWS_PER_TILE = N_PAD // NS

_DOT_KW = dict(preferred_element_type=jnp.float32,
               precision=lax.Precision.HIGHEST)


def _pq_body(x_ref, ws_ref, wd_ref, p_ref, q_ref):
    x = x_ref[...]
    p_ref[...] = lax.dot_general(x, ws_ref[...], (((1,), (0,)), ((), ())),
                                 **_DOT_KW)
    q_ref[...] = lax.dot_general(x, wd_ref[...], (((1,), (0,)), ((), ())),
                                 **_DOT_KW)


_BN = 1000
_pq_call = pl.pallas_call(
    _pq_body,
    grid=(N_NODES // _BN,),
    in_specs=[
        pl.BlockSpec((_BN, D_NODE), lambda i: (i, 0)),
        pl.BlockSpec((D_NODE, D_OUT), lambda i: (0, 0)),
        pl.BlockSpec((D_NODE, D_OUT), lambda i: (0, 0)),
    ],
    out_specs=[
        pl.BlockSpec((_BN, D_OUT), lambda i: (i, 0)),
        pl.BlockSpec((_BN, D_OUT), lambda i: (i, 0)),
    ],
    out_shape=[
        jax.ShapeDtypeStruct((N_NODES, D_OUT), jnp.float32),
        jax.ShapeDtypeStruct((N_NODES, D_OUT), jnp.float32),
    ],
)


def _e1_body(ef_ref, we_ref, b1_ref, e1_ref):
    e1_ref[...] = lax.dot_general(ef_ref[...], we_ref[...],
                                  (((1,), (0,)), ((), ())),
                                  **_DOT_KW) + b1_ref[...]


_BE = 4000
_e1_call = pl.pallas_call(
    _e1_body,
    grid=(N_EDGES // _BE,),
    in_specs=[
        pl.BlockSpec((_BE, D_EDGE), lambda i: (i, 0)),
        pl.BlockSpec((D_EDGE, D_OUT), lambda i: (0, 0)),
        pl.BlockSpec((1, D_OUT), lambda i: (0, 0)),
    ],
    out_specs=pl.BlockSpec((_BE, D_OUT), lambda i: (i, 0)),
    out_shape=jax.ShapeDtypeStruct((N_EDGES, D_OUT), jnp.float32),
)


def _sc_edge_body(p_hbm, q_hbm, e1_hbm, src_hbm, dst_hbm,
                  agg_out, cnt_out,
                  idx_s, idx_d, buf_p, buf_q, buf_e, ones_v, zc_v,
                  agg_sh, cnt_sh, sem_p, sem_q, sem_e):
    cid = lax.axis_index("c")
    sid = lax.axis_index("s")
    wid = sid * NC + cid

    zf = jnp.zeros((LANES,), jnp.float32)
    onef = jnp.ones((LANES,), jnp.float32)

    def _zrow(r, carry):
        for c in range(D_OUT // LANES):
            buf_e[r, pl.ds(c * LANES, LANES)] = zf
        return carry

    lax.fori_loop(0, CHUNK, _zrow, 0)

    def _zcnt(r, carry):
        zc_v[r, pl.ds(0, LANES)] = zf
        ones_v[r, pl.ds(0, LANES)] = onef
        return carry

    lax.fori_loop(0, CHUNK, _zcnt, 0)

    base_row = pl.multiple_of(sid * ROWS_PER_TILE, 8)
    for k in range(ROWS_PER_TILE // CHUNK):
        pltpu.sync_copy(buf_e, agg_sh.at[pl.ds(base_row + k * CHUNK, CHUNK)])
        pltpu.sync_copy(zc_v, cnt_sh.at[pl.ds(base_row + k * CHUNK, CHUNK)])
    plsc.subcore_barrier()

    ebase = wid * EPW

    def _chunk(ci, carry):
        off = pl.multiple_of(ebase + ci * CHUNK, CHUNK)
        cp_e = pltpu.async_copy(e1_hbm.at[pl.ds(off, CHUNK)], buf_e, sem_e)
        pltpu.sync_copy(src_hbm.at[pl.ds(off, CHUNK)], idx_s)
        pltpu.sync_copy(dst_hbm.at[pl.ds(off, CHUNK)], idx_d)
        cp_p = pltpu.async_copy(p_hbm.at[idx_s], buf_p, sem_p)
        cp_q = pltpu.async_copy(q_hbm.at[idx_d], buf_q, sem_q)
        cp_e.wait()
        cp_p.wait()
        cp_q.wait()

        def _row(r, rcarry):
            for c in range(D_OUT // LANES):
                sl = pl.ds(c * LANES, LANES)
                v = buf_p[r, sl] + buf_q[r, sl] + buf_e[r, sl]
                buf_e[r, sl] = jnp.maximum(v, 0.0)
            return rcarry

        lax.fori_loop(0, CHUNK, _row, 0)

        pltpu.sync_copy(buf_e, agg_sh.at[idx_d], add=True)
        pltpu.sync_copy(ones_v, cnt_sh.at[idx_d], add=True)
        return carry

    lax.fori_loop(0, NCHUNKS, _chunk, 0)

    plsc.subcore_barrier()

    pltpu.sync_copy(agg_sh.at[pl.ds(base_row, ROWS_PER_TILE)],
                    agg_out.at[cid, pl.ds(base_row, ROWS_PER_TILE)])
    pltpu.sync_copy(cnt_sh.at[pl.ds(base_row, ROWS_PER_TILE)],
                    cnt_out.at[cid, pl.ds(base_row, ROWS_PER_TILE)])


_sc_edge = pl.kernel(
    _sc_edge_body,
    mesh=plsc.VectorSubcoreMesh(core_axis_name="c", subcore_axis_name="s"),
    compiler_params=pltpu.CompilerParams(use_tc_tiling_on_sc=False),
    out_type=[
        jax.ShapeDtypeStruct((NC, N_PAD, D_OUT), jnp.float32),
        jax.ShapeDtypeStruct((NC, N_PAD, CW), jnp.float32),
    ],
    scratch_types=[
        pltpu.VMEM((CHUNK,), jnp.int32),
        pltpu.VMEM((CHUNK,), jnp.int32),
        pltpu.VMEM((CHUNK, D_OUT), jnp.float32),
        pltpu.VMEM((CHUNK, D_OUT), jnp.float32),
        pltpu.VMEM((CHUNK, D_OUT), jnp.float32),
        pltpu.VMEM((CHUNK, CW), jnp.float32),
        pltpu.VMEM((CHUNK, CW), jnp.float32),
        pltpu.VMEM_SHARED((N_PAD, D_OUT), jnp.float32),
        pltpu.VMEM_SHARED((N_PAD, CW), jnp.float32),
        pltpu.SemaphoreType.DMA,
        pltpu.SemaphoreType.DMA,
        pltpu.SemaphoreType.DMA,
    ],
)


def _out_body(a0_ref, a1_ref, c0_ref, c1_ref, w2_ref, b2_ref, o_ref):
    agg = a0_ref[...] + a1_ref[...]
    deg = c0_ref[...][:, :1] + c1_ref[...][:, :1]
    o_ref[...] = lax.dot_general(agg, w2_ref[...], (((1,), (0,)), ((), ())),
                                 **_DOT_KW) + deg * b2_ref[...]


_out_call = pl.pallas_call(
    _out_body,
    grid=(N_NODES // _BN,),
    in_specs=[
        pl.BlockSpec((_BN, D_OUT), lambda i: (i, 0)),
        pl.BlockSpec((_BN, D_OUT), lambda i: (i, 0)),
        pl.BlockSpec((_BN, CW), lambda i: (i, 0)),
        pl.BlockSpec((_BN, CW), lambda i: (i, 0)),
        pl.BlockSpec((D_OUT, D_OUT), lambda i: (0, 0)),
        pl.BlockSpec((1, D_OUT), lambda i: (0, 0)),
    ],
    out_specs=pl.BlockSpec((_BN, D_OUT), lambda i: (i, 0)),
    out_shape=jax.ShapeDtypeStruct((N_NODES, D_OUT), jnp.float32),
)


def kernel(node_feats, edge_index, edge_feats, W1, b1, W2, b2):
    src = edge_index[0].astype(jnp.int32)
    dst = edge_index[1].astype(jnp.int32)
    p, q = _pq_call(node_feats, W1[:D_NODE], W1[D_NODE:2 * D_NODE])
    e1 = _e1_call(edge_feats, W1[2 * D_NODE:], b1.reshape(1, D_OUT))
    agg2, cnt2 = _sc_edge(p, q, e1, src, dst)
    out = _out_call(agg2[0], agg2[1], cnt2[0], cnt2[1],
                    W2, b2.reshape(1, D_OUT))
    return out
